# baseline (speedup 1.0000x reference)
import jax
import jax.numpy as jnp
from jax import lax
from jax.experimental import pallas as pl
from jax.experimental.pallas import tpu as pltpu
from jax.experimental.pallas import tpu_sc as plsc

END_TOKEN = 2
FULL_WORD_REWARD = 2.0

V = 1_000_000
B, L = 16384, 200
N = B * L
NC, NS = 2, 16
NW = NC * NS
PER_W = N // NW
CHUNK = 10_240
NCHUNKS = PER_W // CHUNK

P1_CHUNK = 8_000
P1_NCHUNKS = V // P1_CHUNK
P1_K = -(-P1_NCHUNKS // NS)


def _body(tw_hbm, trie_hbm, rmap_hbm, out_hbm,
          i0, i1, i2, v0, v1, p_in, p_out, lut, fsh,
          si0, si1, si2, sv0, sv1, so0, so1):
    cid = lax.axis_index("c")
    sid = lax.axis_index("s")
    wid = sid * NC + cid
    base = wid * PER_W
    idx = (i0, i1, i2)
    val = (v0, v1)
    isem = (si0, si1, si2)
    gsem = (sv0, sv1)
    osem = (so0, so1)

    ld = {}
    g = {}
    st = {}

    def start_ld(i):
        ld[i] = pltpu.async_copy(
            tw_hbm.at[pl.ds(base + i * CHUNK, CHUNK)], idx[i % 3], isem[i % 3])

    def start_st(i):
        st[i] = pltpu.async_copy(
            val[i % 2], out_hbm.at[pl.ds(base + i * CHUNK, CHUNK)], osem[i % 2])

    start_ld(0)
    start_ld(1)

    pltpu.sync_copy(rmap_hbm, lut.at[pl.ds(0, 5)])
    lane = lax.iota(jnp.int32, 16)
    lut16 = lut[pl.ds(0, 16)]

    for k in range(P1_K):
        cidx = sid + NS * k

        @pl.when(cidx < P1_NCHUNKS)
        def _():
            off = cidx * P1_CHUNK
            pltpu.sync_copy(trie_hbm.at[pl.ds(off, P1_CHUNK)], p_in)

            def map_vreg(j, carry):
                s = p_in[pl.ds(j * 16, 16)]
                m = jnp.minimum(jnp.maximum(s + 1, 0), 4)
                p_out[pl.ds(j * 16, 16)] = lax.gather(
                    lut16, m[:, None],
                    dimension_numbers=lax.GatherDimensionNumbers(
                        offset_dims=(), collapsed_slice_dims=(0,),
                        start_index_map=(0,)),
                    slice_sizes=(1,),
                    mode=lax.GatherScatterMode.PROMISE_IN_BOUNDS)
                return carry

            lax.fori_loop(0, P1_CHUNK // 16, map_vreg, 0, unroll=8)
            if k == 0:
                @pl.when(sid == 0)
                def _():
                    v = p_out[pl.ds(0, 16)]
                    v = v + jnp.where(lane == END_TOKEN, FULL_WORD_REWARD, 0.0)
                    p_out[pl.ds(0, 16)] = v
            pltpu.sync_copy(p_out, fsh.at[pl.ds(off, P1_CHUNK)])

    plsc.subcore_barrier()

    for i in range(NCHUNKS):
        if i >= 2:
            st[i - 2].wait()
        ld[i].wait()
        g[i] = pltpu.async_copy(fsh.at[idx[i % 3]], val[i % 2], gsem[i % 2])
        if i >= 1:
            g[i - 1].wait()
            start_st(i - 1)
        if i + 2 < NCHUNKS:
            start_ld(i + 2)
    g[NCHUNKS - 1].wait()
    start_st(NCHUNKS - 1)
    st[NCHUNKS - 2].wait()
    st[NCHUNKS - 1].wait()


_call = pl.kernel(
    _body,
    mesh=plsc.VectorSubcoreMesh(core_axis_name="c", subcore_axis_name="s"),
    out_type=jax.ShapeDtypeStruct((N,), jnp.float32),
    scratch_types=[
        pltpu.VMEM((CHUNK,), jnp.int32),
        pltpu.VMEM((CHUNK,), jnp.int32),
        pltpu.VMEM((CHUNK,), jnp.int32),
        pltpu.VMEM((CHUNK,), jnp.float32),
        pltpu.VMEM((CHUNK,), jnp.float32),
        pltpu.VMEM((P1_CHUNK,), jnp.int32),
        pltpu.VMEM((P1_CHUNK,), jnp.float32),
        pltpu.VMEM((128,), jnp.float32),
        pltpu.VMEM_SHARED((V,), jnp.float32),
        pltpu.SemaphoreType.DMA,
        pltpu.SemaphoreType.DMA,
        pltpu.SemaphoreType.DMA,
        pltpu.SemaphoreType.DMA,
        pltpu.SemaphoreType.DMA,
        pltpu.SemaphoreType.DMA,
        pltpu.SemaphoreType.DMA,
    ],
)


@jax.jit
def kernel(token_words, trie_status, reward_mapping_values):
    out = _call(token_words.reshape(N), trie_status, reward_mapping_values)
    return out.reshape(B, L)


# speedup vs baseline: 1.2495x; 1.2495x over previous
import jax
import jax.numpy as jnp
from jax import lax
from jax.experimental import pallas as pl
from jax.experimental.pallas import tpu as pltpu
from jax.experimental.pallas import tpu_sc as plsc

END_TOKEN = 2
FULL_WORD_REWARD = 2.0

V = 1_000_000
B, L = 16384, 200
N = B * L
NC, NS = 2, 16
NW = NC * NS
PER_W = N // NW
CHUNK = 10_240
NCHUNKS = PER_W // CHUNK

VP = 1_007_616
TBLK = VP // 8
TAIL = V - 7 * TBLK
SEG = VP // 16


def _fuse_body(rm_ref, t_in, t_out):
    s = t_in[...]
    m = jnp.minimum(jnp.maximum(s + 1, 0), 4)
    r = jnp.full((TBLK,), rm_ref[0], jnp.float32)
    for k in range(1, 5):
        r = jnp.where(m == k, rm_ref[k], r)
    t_out[...] = r

    @pl.when(pl.program_id(0) == 0)
    def _():
        v = t_out[pl.ds(0, 1024)].reshape(8, 128)
        gi = (lax.broadcasted_iota(jnp.int32, (8, 128), 0) * 128
              + lax.broadcasted_iota(jnp.int32, (8, 128), 1))
        v = v + jnp.where(gi == END_TOKEN, FULL_WORD_REWARD, 0.0)
        t_out[pl.ds(0, 1024)] = v.reshape(1024)


_fuse = pl.pallas_call(
    _fuse_body,
    grid=(VP // TBLK,),
    in_specs=[
        pl.BlockSpec(memory_space=pltpu.SMEM),
        pl.BlockSpec((TBLK,), lambda i: (i,)),
    ],
    out_specs=pl.BlockSpec((TBLK,), lambda i: (i,)),
    out_shape=jax.ShapeDtypeStruct((VP,), jnp.float32),
)


def _gather_body(tw_hbm, fused_hbm, out_hbm,
                 i0, i1, i2, v0, v1, fsh,
                 si0, si1, si2, sv0, sv1, so0, so1):
    cid = lax.axis_index("c")
    sid = lax.axis_index("s")
    wid = sid * NC + cid
    base = wid * PER_W
    idx = (i0, i1, i2)
    val = (v0, v1)
    isem = (si0, si1, si2)
    gsem = (sv0, sv1)
    osem = (so0, so1)

    ld = {}
    g = {}
    st = {}

    def start_ld(i):
        ld[i] = pltpu.async_copy(
            tw_hbm.at[pl.ds(base + i * CHUNK, CHUNK)], idx[i % 3], isem[i % 3])

    def start_st(i):
        st[i] = pltpu.async_copy(
            val[i % 2], out_hbm.at[pl.ds(base + i * CHUNK, CHUNK)], osem[i % 2])

    start_ld(0)
    start_ld(1)

    pltpu.sync_copy(fused_hbm.at[pl.ds(sid * SEG, SEG)],
                    fsh.at[pl.ds(sid * SEG, SEG)])

    plsc.subcore_barrier()

    for i in range(NCHUNKS):
        if i >= 2:
            st[i - 2].wait()
        ld[i].wait()
        g[i] = pltpu.async_copy(fsh.at[idx[i % 3]], val[i % 2], gsem[i % 2])
        if i >= 1:
            g[i - 1].wait()
            start_st(i - 1)
        if i + 2 < NCHUNKS:
            start_ld(i + 2)
    g[NCHUNKS - 1].wait()
    start_st(NCHUNKS - 1)
    st[NCHUNKS - 2].wait()
    st[NCHUNKS - 1].wait()


_gather_call = pl.kernel(
    _gather_body,
    mesh=plsc.VectorSubcoreMesh(core_axis_name="c", subcore_axis_name="s"),
    out_type=jax.ShapeDtypeStruct((N,), jnp.float32),
    scratch_types=[
        pltpu.VMEM((CHUNK,), jnp.int32),
        pltpu.VMEM((CHUNK,), jnp.int32),
        pltpu.VMEM((CHUNK,), jnp.int32),
        pltpu.VMEM((CHUNK,), jnp.float32),
        pltpu.VMEM((CHUNK,), jnp.float32),
        pltpu.VMEM_SHARED((VP,), jnp.float32),
        pltpu.SemaphoreType.DMA,
        pltpu.SemaphoreType.DMA,
        pltpu.SemaphoreType.DMA,
        pltpu.SemaphoreType.DMA,
        pltpu.SemaphoreType.DMA,
        pltpu.SemaphoreType.DMA,
        pltpu.SemaphoreType.DMA,
    ],
)


@jax.jit
def kernel(token_words, trie_status, reward_mapping_values):
    fused = _fuse(reward_mapping_values, trie_status)
    out = _gather_call(token_words.reshape(N), fused)
    return out.reshape(B, L)
